# baseline (device time: 9369 ns/iter reference)
import jax
import jax.numpy as jnp
from jax import lax
from jax.experimental import pallas as pl
from jax.experimental.pallas import tpu as pltpu

N_DEV = 4


def kernel(x):
    m, n = x.shape
    h = m // 2

    def body(x_hbm_ref, out_hbm_ref, x_vmem, send_ref, s2_ref, r1_ref,
             r2_ref, fin_ref, send_sems, recv_sems, in_sem, out_sems):
        my = lax.axis_index("i")
        s = 1 - 2 * (my % 2)
        pa = (my + s) % N_DEV
        pb = (my - s + N_DEV) % N_DEV

        load = pltpu.make_async_copy(x_hbm_ref, x_vmem, in_sem)
        load.start()

        barrier_sem = pltpu.get_barrier_semaphore()
        for nbr in (pa, pb):
            pl.semaphore_signal(
                barrier_sem,
                inc=1,
                device_id=(nbr,),
                device_id_type=pl.DeviceIdType.MESH,
            )
        load.wait()
        send_ref[...] = x_vmem[...].astype(jnp.bfloat16)
        pl.semaphore_wait(barrier_sem, 2)

        s1a = pltpu.make_async_remote_copy(
            src_ref=send_ref.at[pl.ds(0, h)],
            dst_ref=r1_ref.at[0],
            send_sem=send_sems.at[0],
            recv_sem=recv_sems.at[0],
            device_id=(pa,),
            device_id_type=pl.DeviceIdType.MESH,
        )
        s1b = pltpu.make_async_remote_copy(
            src_ref=send_ref.at[pl.ds(h, h)],
            dst_ref=r1_ref.at[1],
            send_sem=send_sems.at[1],
            recv_sem=recv_sems.at[1],
            device_id=(pb,),
            device_id_type=pl.DeviceIdType.MESH,
        )
        s1a.start()
        s1b.start()

        s1b.wait_recv()
        s2_ref[1] = send_ref[pl.ds(h, h)] + r1_ref[1]
        s2a = pltpu.make_async_remote_copy(
            src_ref=s2_ref.at[1],
            dst_ref=r2_ref.at[1],
            send_sem=send_sems.at[2],
            recv_sem=recv_sems.at[2],
            device_id=(pa,),
            device_id_type=pl.DeviceIdType.MESH,
        )
        s2a.start()

        s1a.wait_recv()
        s2_ref[0] = send_ref[pl.ds(0, h)] + r1_ref[0]
        s2b = pltpu.make_async_remote_copy(
            src_ref=s2_ref.at[0],
            dst_ref=r2_ref.at[0],
            send_sem=send_sems.at[3],
            recv_sem=recv_sems.at[3],
            device_id=(pb,),
            device_id_type=pl.DeviceIdType.MESH,
        )
        s2b.start()

        s2b.wait_recv()
        fin_ref[0] = s2_ref[0] + r2_ref[0]
        st0 = pltpu.make_async_copy(
            fin_ref.at[0], out_hbm_ref.at[pl.ds(0, h)], out_sems.at[0]
        )
        st0.start()
        s2a.wait_recv()
        fin_ref[1] = s2_ref[1] + r2_ref[1]
        st1 = pltpu.make_async_copy(
            fin_ref.at[1], out_hbm_ref.at[pl.ds(h, h)], out_sems.at[1]
        )
        st1.start()
        st0.wait()
        st1.wait()

        for rdma in (s1a, s1b, s2a, s2b):
            rdma.wait_send()

    return pl.pallas_call(
        body,
        out_shape=jax.ShapeDtypeStruct((m, n), jnp.bfloat16),
        in_specs=[pl.BlockSpec(memory_space=pltpu.MemorySpace.HBM)],
        out_specs=pl.BlockSpec(memory_space=pltpu.MemorySpace.HBM),
        scratch_shapes=[
            pltpu.VMEM((m, n), jnp.float32),
            pltpu.VMEM((m, n), jnp.bfloat16),
            pltpu.VMEM((2, h, n), jnp.bfloat16),
            pltpu.VMEM((2, h, n), jnp.bfloat16),
            pltpu.VMEM((2, h, n), jnp.bfloat16),
            pltpu.VMEM((2, h, n), jnp.bfloat16),
            pltpu.SemaphoreType.DMA((4,)),
            pltpu.SemaphoreType.DMA((4,)),
            pltpu.SemaphoreType.DMA,
            pltpu.SemaphoreType.DMA((2,)),
        ],
        compiler_params=pltpu.CompilerParams(collective_id=0),
    )(x)


# device time: 8896 ns/iter; 1.0532x vs baseline; 1.0532x over previous
import jax
import jax.numpy as jnp
from jax import lax
from jax.experimental import pallas as pl
from jax.experimental.pallas import tpu as pltpu

N_DEV = 4


def kernel(x):
    m, n = x.shape
    h = m // 2
    q = h // 2

    CHUNKS = (("a", "b"), ("a", "b"), ("b", "a"), ("b", "a"))

    def body(x_ref, out_ref, send_ref, s2_ref, r1_ref, r2_ref,
             send_sems, recv_sems):
        my = lax.axis_index("i")
        s = 1 - 2 * (my % 2)
        pa = (my + s) % N_DEV
        pb = (my - s + N_DEV) % N_DEV

        barrier_sem = pltpu.get_barrier_semaphore()
        for nbr in (pa, pb):
            pl.semaphore_signal(
                barrier_sem,
                inc=1,
                device_id=(nbr,),
                device_id_type=pl.DeviceIdType.MESH,
            )
        send_ref[...] = x_ref[...].astype(jnp.bfloat16)
        pl.semaphore_wait(barrier_sem, 2)

        dev = {"a": (pa,), "b": (pb,)}

        s1 = []
        for c, (tgt1, _) in enumerate(CHUNKS):
            rdma = pltpu.make_async_remote_copy(
                src_ref=send_ref.at[pl.ds(c * q, q)],
                dst_ref=r1_ref.at[c],
                send_sem=send_sems.at[c],
                recv_sem=recv_sems.at[c],
                device_id=dev[tgt1],
                device_id_type=pl.DeviceIdType.MESH,
            )
            rdma.start()
            s1.append(rdma)

        s2 = {}
        for c in (0, 2, 1, 3):
            tgt2 = CHUNKS[c][1]
            s1[c].wait_recv()
            s2_ref[c] = send_ref[pl.ds(c * q, q)] + r1_ref[c]
            rdma = pltpu.make_async_remote_copy(
                src_ref=s2_ref.at[c],
                dst_ref=r2_ref.at[c],
                send_sem=send_sems.at[4 + c],
                recv_sem=recv_sems.at[4 + c],
                device_id=dev[tgt2],
                device_id_type=pl.DeviceIdType.MESH,
            )
            rdma.start()
            s2[c] = rdma

        for c in (0, 2, 1, 3):
            s2[c].wait_recv()
            out_ref[pl.ds(c * q, q), :] = s2_ref[c] + r2_ref[c]

        for rdma in list(s1) + list(s2.values()):
            rdma.wait_send()

    return pl.pallas_call(
        body,
        out_shape=jax.ShapeDtypeStruct((m, n), jnp.bfloat16),
        in_specs=[pl.BlockSpec(memory_space=pltpu.VMEM)],
        out_specs=pl.BlockSpec(memory_space=pltpu.VMEM),
        scratch_shapes=[
            pltpu.VMEM((m, n), jnp.bfloat16),
            pltpu.VMEM((4, q, n), jnp.bfloat16),
            pltpu.VMEM((4, q, n), jnp.bfloat16),
            pltpu.VMEM((4, q, n), jnp.bfloat16),
            pltpu.SemaphoreType.DMA((8,)),
            pltpu.SemaphoreType.DMA((8,)),
        ],
        compiler_params=pltpu.CompilerParams(collective_id=0),
    )(x)


# device time: 8875 ns/iter; 1.0557x vs baseline; 1.0024x over previous
import jax
import jax.numpy as jnp
from jax import lax
from jax.experimental import pallas as pl
from jax.experimental.pallas import tpu as pltpu

N_DEV = 4


def kernel(x):
    m, n = x.shape
    h = m // 2
    K = 4
    q = h // K
    NC = 2 * K

    CHUNKS = tuple(("a", "b") if c < K else ("b", "a") for c in range(NC))
    ORDER = tuple(c for i in range(K) for c in (i, K + i))

    def body(x_ref, out_ref, send_ref, s2_ref, r1_ref, r2_ref,
             send_sems, recv_sems):
        my = lax.axis_index("i")
        s = 1 - 2 * (my % 2)
        pa = (my + s) % N_DEV
        pb = (my - s + N_DEV) % N_DEV

        barrier_sem = pltpu.get_barrier_semaphore()
        for nbr in (pa, pb):
            pl.semaphore_signal(
                barrier_sem,
                inc=1,
                device_id=(nbr,),
                device_id_type=pl.DeviceIdType.MESH,
            )
        send_ref[...] = x_ref[...].astype(jnp.bfloat16)
        pl.semaphore_wait(barrier_sem, 2)

        dev = {"a": (pa,), "b": (pb,)}

        s1 = []
        for c, (tgt1, _) in enumerate(CHUNKS):
            rdma = pltpu.make_async_remote_copy(
                src_ref=send_ref.at[pl.ds(c * q, q)],
                dst_ref=r1_ref.at[c],
                send_sem=send_sems.at[c],
                recv_sem=recv_sems.at[c],
                device_id=dev[tgt1],
                device_id_type=pl.DeviceIdType.MESH,
            )
            rdma.start()
            s1.append(rdma)

        s2 = {}
        for c in ORDER:
            tgt2 = CHUNKS[c][1]
            s1[c].wait_recv()
            s2_ref[c] = send_ref[pl.ds(c * q, q)] + r1_ref[c]
            rdma = pltpu.make_async_remote_copy(
                src_ref=s2_ref.at[c],
                dst_ref=r2_ref.at[c],
                send_sem=send_sems.at[NC + c],
                recv_sem=recv_sems.at[NC + c],
                device_id=dev[tgt2],
                device_id_type=pl.DeviceIdType.MESH,
            )
            rdma.start()
            s2[c] = rdma

        for c in ORDER:
            s2[c].wait_recv()
            out_ref[pl.ds(c * q, q), :] = s2_ref[c] + r2_ref[c]

        for rdma in list(s1) + list(s2.values()):
            rdma.wait_send()

    return pl.pallas_call(
        body,
        out_shape=jax.ShapeDtypeStruct((m, n), jnp.bfloat16),
        in_specs=[pl.BlockSpec(memory_space=pltpu.VMEM)],
        out_specs=pl.BlockSpec(memory_space=pltpu.VMEM),
        scratch_shapes=[
            pltpu.VMEM((m, n), jnp.bfloat16),
            pltpu.VMEM((NC, q, n), jnp.bfloat16),
            pltpu.VMEM((NC, q, n), jnp.bfloat16),
            pltpu.VMEM((NC, q, n), jnp.bfloat16),
            pltpu.SemaphoreType.DMA((2 * NC,)),
            pltpu.SemaphoreType.DMA((2 * NC,)),
        ],
        compiler_params=pltpu.CompilerParams(collective_id=0),
    )(x)
